# baseline (speedup 1.0000x reference)
import functools

import numpy as np
import jax
import jax.numpy as jnp
from jax import lax
from jax.experimental import pallas as pl
from jax.experimental.pallas import tpu as pltpu
from jax.experimental.pallas import tpu_sc as plsc

N = 10000
E = 320000
D = 128

NC = 2
NS = 16
NW = NC * NS

BATCH = 128
EPAD = 327680
BPT = EPAD // (NW * BATCH)
NPAD = 10240
RT = NPAD // NS
DEG_W = 16

_mesh = plsc.VectorSubcoreMesh(core_axis_name="c", subcore_axis_name="s",
                               num_cores=NC, num_subcores=NS)


HB = 16
CHUNKS = BPT // HB

PERM = np.zeros(D, np.int32)
for _g in range(D // 32):
    for _k in range(16):
        PERM[32 * _g + _k] = 32 * _g + 2 * _k
        PERM[32 * _g + 16 + _k] = 32 * _g + 2 * _k + 1
IPERM = np.argsort(PERM)


def _agg_body(zw_hbm, src_hbm, dst_hbm, zeros_hbm, out_hbm,
              idx_src, idx_dst, wbuf, rows0, rows1,
              gsem, ssem0, ssem1, zsem, agg_sh):
    cid = lax.axis_index("c")
    sid = lax.axis_index("s")
    wid = sid * NC + cid

    pltpu.async_copy(zeros_hbm.at[pl.ds(sid * RT, RT), :],
                     agg_sh.at[pl.ds(sid * RT, RT), :], zsem)

    fbufs = (rows0, rows1)
    ssems = (ssem0, ssem1)
    mask_hi = jnp.full((16,), -65536, jnp.int32)
    sh16 = jnp.full((16,), 16, jnp.int32)

    for chunk in range(CHUNKS):
        if chunk > 0:
            for b in range(2):
                pltpu.make_async_copy(fbufs[b], agg_sh.at[idx_dst.at[0]],
                                      ssems[b]).wait()
        base = wid * BPT + chunk * HB
        pltpu.sync_copy(src_hbm.at[pl.ds(base, HB), :], idx_src)
        pltpu.sync_copy(dst_hbm.at[pl.ds(base, HB), :], idx_dst)
        pltpu.async_copy(zw_hbm.at[idx_src.at[0]], wbuf, gsem)
        if chunk == 0:
            pltpu.make_async_copy(zeros_hbm.at[pl.ds(sid * RT, RT), :],
                                  agg_sh.at[pl.ds(sid * RT, RT), :], zsem).wait()
            plsc.subcore_barrier()

        @pl.loop(0, HB, step=2)
        def _(j):
            for b in range(2):
                jj = j + b
                pltpu.make_async_copy(zw_hbm.at[idx_src.at[jj]], wbuf, gsem).wait()

                @pl.when(jj >= 2)
                def _():
                    pltpu.make_async_copy(fbufs[b], agg_sh.at[idx_dst.at[0]],
                                          ssems[b]).wait()

                @pl.loop(0, BATCH, step=16)
                def _(r):
                    for dr in range(16):
                        for c in range(D // 32):
                            w = wbuf[r + dr, pl.ds(c * 16, 16)]
                            ev = lax.bitcast_convert_type(lax.shift_left(w, sh16), jnp.float32)
                            od = lax.bitcast_convert_type(w & mask_hi, jnp.float32)
                            fbufs[b][r + dr, pl.ds(c * 32, 16)] = ev
                            fbufs[b][r + dr, pl.ds(c * 32 + 16, 16)] = od

                @pl.when(jj + 1 < HB)
                def _():
                    pltpu.async_copy(zw_hbm.at[idx_src.at[jj + 1]], wbuf, gsem)

                pltpu.async_copy(fbufs[b], agg_sh.at[idx_dst.at[jj]], ssems[b],
                                 add=True)

    for b in range(2):
        pltpu.make_async_copy(fbufs[b], agg_sh.at[idx_dst.at[0]], ssems[b]).wait()

    plsc.subcore_barrier()

    pltpu.sync_copy(agg_sh.at[pl.ds(sid * RT, RT), :],
                    out_hbm.at[cid, pl.ds(sid * RT, RT), :])


_agg_out_type = jax.ShapeDtypeStruct((NC, NPAD, D), jnp.float32)
_agg_scratch = [
    pltpu.VMEM((HB, BATCH), jnp.int32),
    pltpu.VMEM((HB, BATCH), jnp.int32),
    pltpu.VMEM((BATCH, D // 2), jnp.int32),
    pltpu.VMEM((BATCH, D), jnp.float32),
    pltpu.VMEM((BATCH, D), jnp.float32),
    pltpu.SemaphoreType.DMA,
    pltpu.SemaphoreType.DMA,
    pltpu.SemaphoreType.DMA,
    pltpu.SemaphoreType.DMA,
    pltpu.VMEM_SHARED((NPAD, D), jnp.float32),
]

_agg = pl.kernel(_agg_body, out_type=_agg_out_type, mesh=_mesh,
                 scratch_types=_agg_scratch,
                 compiler_params=pltpu.CompilerParams(use_tc_tiling_on_sc=False),
                 name="sage_agg")


def _deg_body(dst_hbm, zeros_hbm, out_hbm, idx_dst, ones, deg_sh, zsem):
    cid = lax.axis_index("c")
    sid = lax.axis_index("s")
    wid = sid * NC + cid

    pltpu.async_copy(zeros_hbm.at[pl.ds(sid * RT, RT), :],
                     deg_sh.at[pl.ds(sid * RT, RT), :], zsem)

    one16 = jnp.full((16,), 1.0, jnp.float32)

    @pl.loop(0, BATCH)
    def _(r):
        for k in range(D // 16):
            ones[r, pl.ds(k * 16, 16)] = one16

    pltpu.make_async_copy(zeros_hbm.at[pl.ds(sid * RT, RT), :],
                          deg_sh.at[pl.ds(sid * RT, RT), :], zsem).wait()

    plsc.subcore_barrier()

    base = wid * BPT
    pltpu.sync_copy(dst_hbm.at[pl.ds(base, BPT), :], idx_dst)

    @pl.loop(0, BPT, step=4)
    def _(j):
        for k in range(4):
            pltpu.async_copy(ones, deg_sh.at[idx_dst.at[j + k]], zsem, add=True)
        for k in range(4):
            pltpu.make_async_copy(ones, deg_sh.at[idx_dst.at[j + k]], zsem).wait()

    plsc.subcore_barrier()

    pltpu.sync_copy(deg_sh.at[pl.ds(sid * RT, RT), :],
                    out_hbm.at[cid, pl.ds(sid * RT, RT), :])


_deg = pl.kernel(
    _deg_body,
    out_type=jax.ShapeDtypeStruct((NC, NPAD, D), jnp.float32),
    mesh=_mesh,
    scratch_types=[
        pltpu.VMEM((BPT, BATCH), jnp.int32),
        pltpu.VMEM((BATCH, D), jnp.float32),
        pltpu.VMEM_SHARED((NPAD, D), jnp.float32),
        pltpu.SemaphoreType.DMA,
    ],
    name="sage_deg",
)


BM = 1000


def _mm_body(h_ref, ws_ref, wn_ref, b_ref, pre_ref, z_ref):
    h = h_ref[...]
    pre_ref[...] = jnp.dot(h, ws_ref[...], preferred_element_type=jnp.float32) + b_ref[...]
    z_ref[...] = jnp.dot(h, wn_ref[...], preferred_element_type=jnp.float32).astype(jnp.bfloat16)


_w_spec = pl.BlockSpec((D, D), lambda i: (0, 0))
_b_spec = pl.BlockSpec((1, D), lambda i: (0, 0))
_row_spec = pl.BlockSpec((BM, D), lambda i: (i, 0))
_parts_spec = pl.BlockSpec((NC, BM, D), lambda i: (0, i, 0))
_deg_spec = pl.BlockSpec((NC, BM, D), lambda i: (0, i, 0))

_mm = pl.pallas_call(
    _mm_body,
    grid=(N // BM,),
    in_specs=[_row_spec, _w_spec, _w_spec, _b_spec],
    out_specs=[_row_spec, _row_spec],
    out_shape=[jax.ShapeDtypeStruct((N, D), jnp.float32),
               jax.ShapeDtypeStruct((N, D), jnp.bfloat16)],
)


def _combine_h(pre_ref, parts_ref, deg_ref, relu):
    p = parts_ref[...]
    dg = deg_ref[...]
    deg = dg[0, :, :1] + dg[1, :, :1]
    inv = 1.0 / jnp.maximum(deg, 1.0)
    h = pre_ref[...] + (p[0] + p[1]) * inv
    if relu:
        h = jnp.maximum(h, 0.0)
    return h


def _cmb_mm_body(pre_ref, parts_ref, deg_ref, ws_ref, wn_ref, b_ref, pre_o, z_o):
    h = _combine_h(pre_ref, parts_ref, deg_ref, relu=True)
    pre_o[...] = jnp.dot(h, ws_ref[...], preferred_element_type=jnp.float32) + b_ref[...]
    z_o[...] = jnp.dot(h, wn_ref[...], preferred_element_type=jnp.float32).astype(jnp.bfloat16)


_cmb_mm = pl.pallas_call(
    _cmb_mm_body,
    grid=(N // BM,),
    in_specs=[_row_spec, _parts_spec, _deg_spec, _w_spec, _w_spec, _b_spec],
    out_specs=[_row_spec, _row_spec],
    out_shape=[jax.ShapeDtypeStruct((N, D), jnp.float32),
               jax.ShapeDtypeStruct((N, D), jnp.bfloat16)],
)


def _final_body(pre_ref, parts_ref, deg_ref, out_ref):
    out_ref[...] = _combine_h(pre_ref, parts_ref, deg_ref, relu=False)


_final = pl.pallas_call(
    _final_body,
    grid=(N // BM,),
    in_specs=[_row_spec, _parts_spec, _deg_spec],
    out_specs=_row_spec,
    out_shape=jax.ShapeDtypeStruct((N, D), jnp.float32),
)


def _pack(z16):
    return jax.lax.bitcast_convert_type(z16.reshape(N, D // 2, 2), jnp.int32)


def kernel(in_feat, edge_index, Ws1, Wn1, b1, Ws2, Wn2, b2, Ws3, Wn3, b3,
           Ws4, Wn4, b4, Ws5, Wn5, b5):
    src = edge_index[0]
    dst = edge_index[1]
    pad = EPAD - E
    pad_src = (jnp.arange(pad, dtype=jnp.int32) * 37) % N
    pad_dst = N + (jnp.arange(pad, dtype=jnp.int32) % (NPAD - N))
    src2d = jnp.concatenate([src, pad_src]).reshape(EPAD // BATCH, BATCH)
    dst2d = jnp.concatenate([dst, pad_dst]).reshape(EPAD // BATCH, BATCH)

    perm = jnp.asarray(PERM)
    iperm = jnp.asarray(IPERM)
    Ws1p, b1p = Ws1[:, perm], b1[perm].reshape(1, D)
    Wn1p = Wn1
    mids = []
    for Wsi, Wni, bi in ((Ws2, Wn2, b2), (Ws3, Wn3, b3), (Ws5, Wn5, b5), (Ws4, Wn4, b4)):
        mids.append((Wsi[perm][:, perm], Wni[perm, :], bi[perm].reshape(1, D)))

    zeros = jnp.zeros((NPAD, D), jnp.float32)
    degp = _deg(dst2d, zeros)
    pre, z16 = _mm(in_feat, Ws1p, Wn1p, b1p)
    parts = _agg(_pack(z16), src2d, dst2d, zeros)
    for Wsp, Wnp, bp in mids:
        pre, z16 = _cmb_mm(pre, parts, degp, Wsp, Wnp, bp)
        parts = _agg(_pack(z16), src2d, dst2d, zeros)
    out_p = _final(pre, parts, degp)
    return out_p[:, iperm]


# speedup vs baseline: 2.1619x; 2.1619x over previous
import functools

import jax
import jax.numpy as jnp
from jax import lax
from jax.experimental import pallas as pl
from jax.experimental.pallas import tpu as pltpu
from jax.experimental.pallas import tpu_sc as plsc

N = 10000
E = 320000
D = 128

NC = 2
NS = 16
NW = NC * NS

BATCH = 128
EPAD = 327680
BPT = EPAD // (NW * BATCH)
NPAD = 10240
RT = NPAD // NS
DEG_W = 16

_mesh = plsc.VectorSubcoreMesh(core_axis_name="c", subcore_axis_name="s",
                               num_cores=NC, num_subcores=NS)


def _agg_body(z_hbm, src_hbm, dst_hbm, zeros_hbm, out_hbm,
              idx_src, idx_dst, rows0, rows1,
              agg_sh, gsem0, gsem1, ssem0, ssem1, zsem):
    cid = lax.axis_index("c")
    sid = lax.axis_index("s")
    wid = sid * NC + cid

    pltpu.async_copy(zeros_hbm.at[pl.ds(sid * RT, RT), :],
                     agg_sh.at[pl.ds(sid * RT, RT), :], zsem)

    HB = BPT // 2
    bufs = ((rows0, gsem0, ssem0), (rows1, gsem1, ssem1))
    for half in range(2):
        base = wid * BPT + half * HB
        pltpu.sync_copy(src_hbm.at[pl.ds(base, HB), :], idx_src)
        pltpu.sync_copy(dst_hbm.at[pl.ds(base, HB), :], idx_dst)

        pltpu.async_copy(z_hbm.at[idx_src.at[0]], rows0, gsem0)

        if half == 0:
            pltpu.make_async_copy(zeros_hbm.at[pl.ds(sid * RT, RT), :],
                                  agg_sh.at[pl.ds(sid * RT, RT), :], zsem).wait()
            plsc.subcore_barrier()

        @pl.loop(0, HB, step=2)
        def _(j):
            for b in range(2):
                jj = j + b
                rows_b, gsem_b, _ = bufs[b]
                rows_o, gsem_o, ssem_o = bufs[1 - b]
                pltpu.make_async_copy(z_hbm.at[idx_src.at[jj]], rows_b, gsem_b).wait()
                pltpu.async_copy(rows_b, agg_sh.at[idx_dst.at[jj]], bufs[b][2], add=True)
                @pl.when(jj > 0)
                def _():
                    pltpu.make_async_copy(rows_o, agg_sh.at[idx_dst.at[jj - 1]], ssem_o).wait()

                @pl.when(jj + 1 < HB)
                def _():
                    pltpu.async_copy(z_hbm.at[idx_src.at[jj + 1]], rows_o, gsem_o)

        pltpu.make_async_copy(rows1, agg_sh.at[idx_dst.at[HB - 1]], ssem1).wait()

    plsc.subcore_barrier()

    pltpu.sync_copy(agg_sh.at[pl.ds(sid * RT, RT), :],
                    out_hbm.at[cid, pl.ds(sid * RT, RT), :])


_agg_out_type = jax.ShapeDtypeStruct((NC, NPAD, D), jnp.float32)
_agg_scratch = [
    pltpu.VMEM((BPT // 2, BATCH), jnp.int32),
    pltpu.VMEM((BPT // 2, BATCH), jnp.int32),
    pltpu.VMEM((BATCH, D), jnp.float32),
    pltpu.VMEM((BATCH, D), jnp.float32),
    pltpu.VMEM_SHARED((NPAD, D), jnp.float32),
    pltpu.SemaphoreType.DMA,
    pltpu.SemaphoreType.DMA,
    pltpu.SemaphoreType.DMA,
    pltpu.SemaphoreType.DMA,
    pltpu.SemaphoreType.DMA,
]

_agg = pl.kernel(_agg_body, out_type=_agg_out_type, mesh=_mesh,
                 scratch_types=_agg_scratch, name="sage_agg")


def _deg_body(dst_hbm, zeros_hbm, out_hbm, idx_dst, ones, deg_sh, zsem):
    cid = lax.axis_index("c")
    sid = lax.axis_index("s")
    wid = sid * NC + cid

    pltpu.async_copy(zeros_hbm.at[pl.ds(sid * RT, RT), :],
                     deg_sh.at[pl.ds(sid * RT, RT), :], zsem)

    one16 = jnp.full((16,), 1.0, jnp.float32)

    @pl.loop(0, BATCH)
    def _(r):
        for k in range(D // 16):
            ones[r, pl.ds(k * 16, 16)] = one16

    pltpu.make_async_copy(zeros_hbm.at[pl.ds(sid * RT, RT), :],
                          deg_sh.at[pl.ds(sid * RT, RT), :], zsem).wait()

    plsc.subcore_barrier()

    base = wid * BPT
    pltpu.sync_copy(dst_hbm.at[pl.ds(base, BPT), :], idx_dst)

    @pl.loop(0, BPT, step=4)
    def _(j):
        for k in range(4):
            pltpu.async_copy(ones, deg_sh.at[idx_dst.at[j + k]], zsem, add=True)
        for k in range(4):
            pltpu.make_async_copy(ones, deg_sh.at[idx_dst.at[j + k]], zsem).wait()

    plsc.subcore_barrier()

    pltpu.sync_copy(deg_sh.at[pl.ds(sid * RT, RT), :],
                    out_hbm.at[cid, pl.ds(sid * RT, RT), :])


_deg = pl.kernel(
    _deg_body,
    out_type=jax.ShapeDtypeStruct((NC, NPAD, D), jnp.float32),
    mesh=_mesh,
    scratch_types=[
        pltpu.VMEM((BPT, BATCH), jnp.int32),
        pltpu.VMEM((BATCH, D), jnp.float32),
        pltpu.VMEM_SHARED((NPAD, D), jnp.float32),
        pltpu.SemaphoreType.DMA,
    ],
    name="sage_deg",
)


BM = 1000


def _mm_body(h_ref, ws_ref, wn_ref, b_ref, pre_ref, z_ref):
    h = h_ref[...]
    pre_ref[...] = jnp.dot(h, ws_ref[...], preferred_element_type=jnp.float32) + b_ref[...]
    z_ref[...] = jnp.dot(h, wn_ref[...], preferred_element_type=jnp.float32)


_w_spec = pl.BlockSpec((D, D), lambda i: (0, 0))
_b_spec = pl.BlockSpec((1, D), lambda i: (0, 0))
_row_spec = pl.BlockSpec((BM, D), lambda i: (i, 0))
_parts_spec = pl.BlockSpec((NC, BM, D), lambda i: (0, i, 0))
_deg_spec = pl.BlockSpec((NC, BM, D), lambda i: (0, i, 0))

_mm = pl.pallas_call(
    _mm_body,
    grid=(N // BM,),
    in_specs=[_row_spec, _w_spec, _w_spec, _b_spec],
    out_specs=[_row_spec, _row_spec],
    out_shape=[jax.ShapeDtypeStruct((N, D), jnp.float32)] * 2,
)


def _combine_h(pre_ref, parts_ref, deg_ref, relu):
    p = parts_ref[...]
    dg = deg_ref[...]
    deg = dg[0, :, :1] + dg[1, :, :1]
    inv = 1.0 / jnp.maximum(deg, 1.0)
    h = pre_ref[...] + (p[0] + p[1]) * inv
    if relu:
        h = jnp.maximum(h, 0.0)
    return h


def _cmb_mm_body(pre_ref, parts_ref, deg_ref, ws_ref, wn_ref, b_ref, pre_o, z_o):
    h = _combine_h(pre_ref, parts_ref, deg_ref, relu=True)
    pre_o[...] = jnp.dot(h, ws_ref[...], preferred_element_type=jnp.float32) + b_ref[...]
    z_o[...] = jnp.dot(h, wn_ref[...], preferred_element_type=jnp.float32)


_cmb_mm = pl.pallas_call(
    _cmb_mm_body,
    grid=(N // BM,),
    in_specs=[_row_spec, _parts_spec, _deg_spec, _w_spec, _w_spec, _b_spec],
    out_specs=[_row_spec, _row_spec],
    out_shape=[jax.ShapeDtypeStruct((N, D), jnp.float32)] * 2,
)


def _final_body(pre_ref, parts_ref, deg_ref, out_ref):
    out_ref[...] = _combine_h(pre_ref, parts_ref, deg_ref, relu=False)


_final = pl.pallas_call(
    _final_body,
    grid=(N // BM,),
    in_specs=[_row_spec, _parts_spec, _deg_spec],
    out_specs=_row_spec,
    out_shape=jax.ShapeDtypeStruct((N, D), jnp.float32),
)


def kernel(in_feat, edge_index, Ws1, Wn1, b1, Ws2, Wn2, b2, Ws3, Wn3, b3,
           Ws4, Wn4, b4, Ws5, Wn5, b5):
    src = edge_index[0]
    dst = edge_index[1]
    pad = EPAD - E
    pad_src = (jnp.arange(pad, dtype=jnp.int32) * 37) % N
    pad_dst = N + (jnp.arange(pad, dtype=jnp.int32) % (NPAD - N))
    src2d = jnp.concatenate([src, pad_src]).reshape(EPAD // BATCH, BATCH)
    dst2d = jnp.concatenate([dst, pad_dst]).reshape(EPAD // BATCH, BATCH)

    b1r, b2r, b3r, b4r, b5r = (b.reshape(1, D) for b in (b1, b2, b3, b4, b5))

    zeros = jnp.zeros((NPAD, D), jnp.float32)
    degp = _deg(dst2d, zeros)
    pre, z = _mm(in_feat, Ws1, Wn1, b1r)
    parts = _agg(z, src2d, dst2d, zeros)
    for Ws, Wn, br in ((Ws2, Wn2, b2r), (Ws3, Wn3, b3r), (Ws5, Wn5, b5r), (Ws4, Wn4, b4r)):
        pre, z = _cmb_mm(pre, parts, degp, Ws, Wn, br)
        parts = _agg(z, src2d, dst2d, zeros)
    return _final(pre, parts, degp)
